# baseline (speedup 1.0000x reference)
import functools

import jax
import jax.numpy as jnp
from jax import lax
from jax.experimental import pallas as pl
from jax.experimental.pallas import tpu as pltpu
from jax.experimental.pallas import tpu_sc as plsc

N_ATOMS = 500000
NUM_GRAPHS = 8192
MAX_Z = 89

_NC = 2
_NW = 16
_SUB = 977
_CHUNK = _SUB * 16
_UNROLL = 8
_OUTER = _SUB // _UNROLL
_TAIL = _SUB - _OUTER * _UNROLL
_LAST_REAL = N_ATOMS - (_NC * _NW - 1) * _CHUNK
_G_PAD = NUM_GRAPHS + 16
_STRIPE = NUM_GRAPHS // _NW


def _main_body(prop_hbm, nt_hbm, sg_hbm, out_hbm,
               prop_v, nt_v, sg_v, acc_v, tmp_v, stripe_v, shared, sem):
    cid = lax.axis_index("c")
    sid = lax.axis_index("s")
    wid = cid * _NW + sid
    base = wid * _CHUNK
    last = _NC * _NW - 1

    @pl.when(wid != last)
    def _full_dma():
        cp0 = pltpu.async_copy(nt_hbm.at[pl.ds(base, _CHUNK)], nt_v, sem)
        cp1 = pltpu.async_copy(sg_hbm.at[pl.ds(base, _CHUNK)], sg_v, sem)
        cp0.wait()
        cp1.wait()

    @pl.when(wid == last)
    def _tail_dma():
        cp0 = pltpu.async_copy(nt_hbm.at[pl.ds(base, _LAST_REAL)],
                               nt_v.at[pl.ds(0, _LAST_REAL)], sem)
        cp1 = pltpu.async_copy(sg_hbm.at[pl.ds(base, _LAST_REAL)],
                               sg_v.at[pl.ds(0, _LAST_REAL)], sem)
        npad = _CHUNK - _LAST_REAL
        dead = jnp.full((16,), NUM_GRAPHS, jnp.int32)
        nul = jnp.zeros((16,), jnp.int32)

        def fill(i, c):
            sg_v[pl.ds(_LAST_REAL + i * 16, 16)] = dead
            nt_v[pl.ds(_LAST_REAL + i * 16, 16)] = nul
            return c
        lax.fori_loop(0, npad // 16, fill, 0)
        cp0.wait()
        cp1.wait()

    pltpu.sync_copy(prop_hbm, prop_v)
    zeros = jnp.zeros((16,), jnp.float32)

    def zero_body(i, c):
        acc_v[pl.ds(i * 16, 16)] = zeros
        return c
    lax.fori_loop(0, _G_PAD // 16, zero_body, 0)

    lane_base = lax.iota(jnp.int32, 16) * _SUB

    def step_chain(js, run, cur):
        nts = [plsc.load_gather(nt_v, [lane_base + j]) for j in js]
        sgs = [plsc.load_gather(sg_v, [lane_base + j]) for j in js]
        vals = [plsc.load_gather(prop_v, [nt]) for nt in nts]
        for sg, val in zip(sgs, vals):
            m = jnp.not_equal(sg, cur)
            plsc.addupdate_scatter(acc_v, [cur], run, mask=m)
            run = jnp.where(m, val, run + val)
            cur = sg
        return run, cur

    def body(j, carry):
        run, cur = carry
        j0 = j * _UNROLL
        return step_chain([j0 + u for u in range(_UNROLL)], run, cur)

    run0 = zeros
    cur0 = jnp.full((16,), NUM_GRAPHS, jnp.int32)
    run, cur = lax.fori_loop(0, _OUTER, body, (run0, cur0))
    run, cur = step_chain(
        [_OUTER * _UNROLL + t for t in range(_TAIL)], run, cur)

    lane_ids = lax.iota(jnp.int32, 16)
    for l in range(16):
        plsc.addupdate_scatter(acc_v, [cur], run,
                               mask=jnp.equal(lane_ids, l))

    pltpu.sync_copy(acc_v.at[pl.ds(0, NUM_GRAPHS)], shared.at[sid])
    plsc.subcore_barrier()

    sbase = sid * _STRIPE
    pltpu.sync_copy(shared.at[:, pl.ds(sbase, _STRIPE)], tmp_v)

    def zs(i, c):
        stripe_v[pl.ds(i * 16, 16)] = zeros
        return c
    lax.fori_loop(0, _STRIPE // 16, zs, 0)

    for s in range(_NW):
        def addb(i, c):
            stripe_v[pl.ds(i * 16, 16)] = (
                stripe_v[pl.ds(i * 16, 16)] + tmp_v[s, pl.ds(i * 16, 16)])
            return c
        lax.fori_loop(0, _STRIPE // 16, addb, 0)

    pltpu.sync_copy(stripe_v, out_hbm.at[cid, pl.ds(sbase, _STRIPE)])


def _merge_body(part_hbm, out_hbm, a_v, b_v, sem):
    sid = lax.axis_index("s")
    sbase = sid * _STRIPE
    cpa = pltpu.async_copy(part_hbm.at[0, pl.ds(sbase, _STRIPE)], a_v, sem)
    cpb = pltpu.async_copy(part_hbm.at[1, pl.ds(sbase, _STRIPE)], b_v, sem)
    cpa.wait()
    cpb.wait()

    def addb(i, c):
        a_v[pl.ds(i * 16, 16)] = (
            a_v[pl.ds(i * 16, 16)] + b_v[pl.ds(i * 16, 16)])
        return c
    lax.fori_loop(0, _STRIPE // 16, addb, 0)
    pltpu.sync_copy(a_v, out_hbm.at[pl.ds(sbase, _STRIPE)])


_mesh2 = plsc.VectorSubcoreMesh(
    core_axis_name="c", subcore_axis_name="s", num_cores=2)
_mesh1 = plsc.VectorSubcoreMesh(
    core_axis_name="c", subcore_axis_name="s", num_cores=1)

_main_kernel = functools.partial(
    pl.kernel,
    out_type=jax.ShapeDtypeStruct((_NC, NUM_GRAPHS), jnp.float32),
    mesh=_mesh2,
    compiler_params=pltpu.CompilerParams(needs_layout_passes=False),
    scratch_types=[
        pltpu.VMEM((MAX_Z,), jnp.float32),
        pltpu.VMEM((_CHUNK,), jnp.int32),
        pltpu.VMEM((_CHUNK,), jnp.int32),
        pltpu.VMEM((_G_PAD,), jnp.float32),
        pltpu.VMEM((_NW, _STRIPE), jnp.float32),
        pltpu.VMEM((_STRIPE,), jnp.float32),
        pltpu.VMEM_SHARED((_NW, NUM_GRAPHS), jnp.float32),
        pltpu.SemaphoreType.DMA,
    ],
)(_main_body)

_merge_kernel = functools.partial(
    pl.kernel,
    out_type=jax.ShapeDtypeStruct((NUM_GRAPHS,), jnp.float32),
    mesh=_mesh1,
    compiler_params=pltpu.CompilerParams(needs_layout_passes=False),
    scratch_types=[
        pltpu.VMEM((_STRIPE,), jnp.float32),
        pltpu.VMEM((_STRIPE,), jnp.float32),
        pltpu.SemaphoreType.DMA,
    ],
)(_merge_body)


def kernel(property_offset, node_type, segment_ids):
    partials = _main_kernel(property_offset,
                            node_type.astype(jnp.int32),
                            segment_ids.astype(jnp.int32))
    return _merge_kernel(partials)


# speedup vs baseline: 1.0755x; 1.0755x over previous
import functools

import jax
import jax.numpy as jnp
from jax import lax
from jax.experimental import pallas as pl
from jax.experimental.pallas import tpu as pltpu
from jax.experimental.pallas import tpu_sc as plsc

N_ATOMS = 500000
NUM_GRAPHS = 8192
MAX_Z = 89

_NW = 16
_SUB = 1955
_CHUNK = _SUB * 16
_UNROLL = 8
_OUTER = _SUB // _UNROLL
_TAIL = _SUB - _OUTER * _UNROLL
_LAST_REAL = N_ATOMS - 15 * _CHUNK
_G_PAD = NUM_GRAPHS + 16
_STRIPE = NUM_GRAPHS // _NW


def _sc_body(prop_hbm, nt_hbm, sg_hbm, out_hbm,
             prop_v, nt_v, sg_v, acc_v, tmp_v, stripe_v, shared, sem):
    sid = lax.axis_index("s")
    base = sid * _CHUNK
    last = _NW - 1

    @pl.when(sid != last)
    def _full_dma():
        cp0 = pltpu.async_copy(nt_hbm.at[pl.ds(base, _CHUNK)], nt_v, sem)
        cp1 = pltpu.async_copy(sg_hbm.at[pl.ds(base, _CHUNK)], sg_v, sem)
        cp0.wait()
        cp1.wait()

    @pl.when(sid == last)
    def _tail_dma():
        cp0 = pltpu.async_copy(nt_hbm.at[pl.ds(base, _LAST_REAL)],
                               nt_v.at[pl.ds(0, _LAST_REAL)], sem)
        cp1 = pltpu.async_copy(sg_hbm.at[pl.ds(base, _LAST_REAL)],
                               sg_v.at[pl.ds(0, _LAST_REAL)], sem)
        npad = _CHUNK - _LAST_REAL
        dead = jnp.full((16,), NUM_GRAPHS, jnp.int32)
        nul = jnp.zeros((16,), jnp.int32)

        def fill(i, c):
            sg_v[pl.ds(_LAST_REAL + i * 16, 16)] = dead
            nt_v[pl.ds(_LAST_REAL + i * 16, 16)] = nul
            return c
        lax.fori_loop(0, npad // 16, fill, 0)
        cp0.wait()
        cp1.wait()

    pltpu.sync_copy(prop_hbm, prop_v)
    zeros = jnp.zeros((16,), jnp.float32)

    def zero_body(i, c):
        acc_v[pl.ds(i * 16, 16)] = zeros
        return c
    lax.fori_loop(0, _G_PAD // 16, zero_body, 0)

    lane_base = lax.iota(jnp.int32, 16) * _SUB

    def step_chain(js, run, cur):
        nts = [plsc.load_gather(nt_v, [lane_base + j]) for j in js]
        sgs = [plsc.load_gather(sg_v, [lane_base + j]) for j in js]
        vals = [plsc.load_gather(prop_v, [nt]) for nt in nts]
        for sg, val in zip(sgs, vals):
            m = jnp.not_equal(sg, cur)
            plsc.addupdate_scatter(acc_v, [cur], run, mask=m)
            run = jnp.where(m, val, run + val)
            cur = sg
        return run, cur

    def body(j, carry):
        run, cur = carry
        j0 = j * _UNROLL
        return step_chain([j0 + u for u in range(_UNROLL)], run, cur)

    run0 = zeros
    cur0 = jnp.full((16,), NUM_GRAPHS, jnp.int32)
    run, cur = lax.fori_loop(0, _OUTER, body, (run0, cur0))
    run, cur = step_chain(
        [_OUTER * _UNROLL + t for t in range(_TAIL)], run, cur)

    lane_ids = lax.iota(jnp.int32, 16)
    for l in range(16):
        plsc.addupdate_scatter(acc_v, [cur], run,
                               mask=jnp.equal(lane_ids, l))

    pltpu.sync_copy(acc_v.at[pl.ds(0, NUM_GRAPHS)], shared.at[sid])
    plsc.subcore_barrier()

    sbase = sid * _STRIPE
    pltpu.sync_copy(shared.at[:, pl.ds(sbase, _STRIPE)], tmp_v)

    def zs(i, c):
        stripe_v[pl.ds(i * 16, 16)] = zeros
        return c
    lax.fori_loop(0, _STRIPE // 16, zs, 0)

    for s in range(_NW):
        def addb(i, c):
            stripe_v[pl.ds(i * 16, 16)] = (
                stripe_v[pl.ds(i * 16, 16)] + tmp_v[s, pl.ds(i * 16, 16)])
            return c
        lax.fori_loop(0, _STRIPE // 16, addb, 0)

    pltpu.sync_copy(stripe_v, out_hbm.at[pl.ds(sbase, _STRIPE)])


_mesh = plsc.VectorSubcoreMesh(
    core_axis_name="c", subcore_axis_name="s", num_cores=1)

_sc_kernel = functools.partial(
    pl.kernel,
    out_type=jax.ShapeDtypeStruct((NUM_GRAPHS,), jnp.float32),
    mesh=_mesh,
    compiler_params=pltpu.CompilerParams(needs_layout_passes=False),
    scratch_types=[
        pltpu.VMEM((MAX_Z,), jnp.float32),
        pltpu.VMEM((_CHUNK,), jnp.int32),
        pltpu.VMEM((_CHUNK,), jnp.int32),
        pltpu.VMEM((_G_PAD,), jnp.float32),
        pltpu.VMEM((_NW, _STRIPE), jnp.float32),
        pltpu.VMEM((_STRIPE,), jnp.float32),
        pltpu.VMEM_SHARED((_NW, NUM_GRAPHS), jnp.float32),
        pltpu.SemaphoreType.DMA,
    ],
)(_sc_body)


def kernel(property_offset, node_type, segment_ids):
    return _sc_kernel(property_offset,
                      node_type.astype(jnp.int32),
                      segment_ids.astype(jnp.int32))


# speedup vs baseline: 1.1311x; 1.0517x over previous
import functools

import jax
import jax.numpy as jnp
from jax import lax
from jax.experimental import pallas as pl
from jax.experimental.pallas import tpu as pltpu
from jax.experimental.pallas import tpu_sc as plsc

N_ATOMS = 500000
NUM_GRAPHS = 8192
MAX_Z = 89

_NW = 16
_SUBP = 977
_PHASE = _SUBP * 16
_CHUNK = 2 * _PHASE
_UNROLL = 8
_OUTER = _SUBP // _UNROLL
_TAIL = _SUBP - _OUTER * _UNROLL
_LAST_REAL = N_ATOMS - (_NW - 1) * _CHUNK
_B_REAL = _LAST_REAL - _PHASE
_G_PAD = NUM_GRAPHS + 16
_STRIPE = NUM_GRAPHS // _NW


def _sc_body(prop_hbm, nt_hbm, sg_hbm, out_hbm,
             prop_v, nt_v, sg_v, acc_v, tmp_v, stripe_v, shared, sem):
    sid = lax.axis_index("s")
    base_a = sid * _PHASE
    base_b = (sid + _NW) * _PHASE
    last = _NW - 1

    cpa0 = pltpu.async_copy(nt_hbm.at[pl.ds(base_a, _PHASE)],
                            nt_v.at[pl.ds(0, _PHASE)], sem)
    cpa1 = pltpu.async_copy(sg_hbm.at[pl.ds(base_a, _PHASE)],
                            sg_v.at[pl.ds(0, _PHASE)], sem)

    b_handles = []

    @pl.when(sid != last)
    def _b_full():
        cp0 = pltpu.async_copy(nt_hbm.at[pl.ds(base_b, _PHASE)],
                               nt_v.at[pl.ds(_PHASE, _PHASE)], sem)
        cp1 = pltpu.async_copy(sg_hbm.at[pl.ds(base_b, _PHASE)],
                               sg_v.at[pl.ds(_PHASE, _PHASE)], sem)
        b_handles.append((cp0, cp1))

    @pl.when(sid == last)
    def _b_tail():
        cp0 = pltpu.async_copy(nt_hbm.at[pl.ds(base_b, _B_REAL)],
                               nt_v.at[pl.ds(_PHASE, _B_REAL)], sem)
        cp1 = pltpu.async_copy(sg_hbm.at[pl.ds(base_b, _B_REAL)],
                               sg_v.at[pl.ds(_PHASE, _B_REAL)], sem)
        npad = _PHASE - _B_REAL
        dead = jnp.full((16,), NUM_GRAPHS, jnp.int32)
        nul = jnp.zeros((16,), jnp.int32)

        def fill(i, c):
            sg_v[pl.ds(_PHASE + _B_REAL + i * 16, 16)] = dead
            nt_v[pl.ds(_PHASE + _B_REAL + i * 16, 16)] = nul
            return c
        lax.fori_loop(0, npad // 16, fill, 0)
        b_handles.append((cp0, cp1))

    pltpu.sync_copy(prop_hbm, prop_v)
    zeros = jnp.zeros((16,), jnp.float32)

    def zero_body(i, c):
        acc_v[pl.ds(i * 16, 16)] = zeros
        return c
    lax.fori_loop(0, _G_PAD // 16, zero_body, 0)

    lane_ids = lax.iota(jnp.int32, 16)

    def run_phase(off):
        lane_base = lane_ids * _SUBP + off

        def step_chain(js, run, cur):
            nts = [plsc.load_gather(nt_v, [lane_base + j]) for j in js]
            sgs = [plsc.load_gather(sg_v, [lane_base + j]) for j in js]
            vals = [plsc.load_gather(prop_v, [nt]) for nt in nts]
            for sg, val in zip(sgs, vals):
                m = jnp.not_equal(sg, cur)
                plsc.addupdate_scatter(acc_v, [cur], run, mask=m)
                run = jnp.where(m, val, run + val)
                cur = sg
            return run, cur

        def body(j, carry):
            run, cur = carry
            j0 = j * _UNROLL
            return step_chain([j0 + u for u in range(_UNROLL)], run, cur)

        run0 = zeros
        cur0 = jnp.full((16,), NUM_GRAPHS, jnp.int32)
        run, cur = lax.fori_loop(0, _OUTER, body, (run0, cur0))
        run, cur = step_chain(
            [_OUTER * _UNROLL + t for t in range(_TAIL)], run, cur)

        for l in range(16):
            plsc.addupdate_scatter(acc_v, [cur], run,
                                   mask=jnp.equal(lane_ids, l))

    cpa0.wait()
    cpa1.wait()
    run_phase(0)

    @pl.when(sid != last)
    def _wait_b_full():
        cp0, cp1 = b_handles[0]
        cp0.wait()
        cp1.wait()

    @pl.when(sid == last)
    def _wait_b_tail():
        cp0, cp1 = b_handles[1]
        cp0.wait()
        cp1.wait()

    run_phase(_PHASE)

    pltpu.sync_copy(acc_v.at[pl.ds(0, NUM_GRAPHS)], shared.at[sid])
    plsc.subcore_barrier()

    sbase = sid * _STRIPE
    pltpu.sync_copy(shared.at[:, pl.ds(sbase, _STRIPE)], tmp_v)

    def zs(i, c):
        stripe_v[pl.ds(i * 16, 16)] = zeros
        return c
    lax.fori_loop(0, _STRIPE // 16, zs, 0)

    for s in range(_NW):
        def addb(i, c):
            stripe_v[pl.ds(i * 16, 16)] = (
                stripe_v[pl.ds(i * 16, 16)] + tmp_v[s, pl.ds(i * 16, 16)])
            return c
        lax.fori_loop(0, _STRIPE // 16, addb, 0)

    pltpu.sync_copy(stripe_v, out_hbm.at[pl.ds(sbase, _STRIPE)])


_mesh = plsc.VectorSubcoreMesh(
    core_axis_name="c", subcore_axis_name="s", num_cores=1)

_sc_kernel = functools.partial(
    pl.kernel,
    out_type=jax.ShapeDtypeStruct((NUM_GRAPHS,), jnp.float32),
    mesh=_mesh,
    compiler_params=pltpu.CompilerParams(needs_layout_passes=False),
    scratch_types=[
        pltpu.VMEM((MAX_Z,), jnp.float32),
        pltpu.VMEM((_CHUNK,), jnp.int32),
        pltpu.VMEM((_CHUNK,), jnp.int32),
        pltpu.VMEM((_G_PAD,), jnp.float32),
        pltpu.VMEM((_NW, _STRIPE), jnp.float32),
        pltpu.VMEM((_STRIPE,), jnp.float32),
        pltpu.VMEM_SHARED((_NW, NUM_GRAPHS), jnp.float32),
        pltpu.SemaphoreType.DMA,
    ],
)(_sc_body)


def kernel(property_offset, node_type, segment_ids):
    return _sc_kernel(property_offset,
                      node_type.astype(jnp.int32),
                      segment_ids.astype(jnp.int32))
